# baseline (speedup 1.0000x reference)
import functools

import jax
import jax.numpy as jnp
from jax import lax
from jax.experimental import pallas as pl
from jax.experimental.pallas import tpu as pltpu
from jax.experimental.pallas import tpu_sc as plsc

R = 16
N = 2048
L = 262144

NUM_CORES = 2
NUM_SUBCORES = 16
NUM_WORKERS = NUM_CORES * NUM_SUBCORES
B = L // NUM_WORKERS
LANES = 16
STEPS = B // LANES


def _sc_body(table_hbm, packed_hbm, outp_hbm, outn_hbm,
             packed_v, idx_v, par_v, sf_v, vals_v, accp_v, accn_v, sem):
    wid = lax.axis_index("s") * NUM_CORES + lax.axis_index("c")
    base = wid * B

    pltpu.sync_copy(packed_hbm.at[pl.ds(base, B)], packed_v)

    def idx_body(i, carry):
        off = i * LANES
        p = packed_v[pl.ds(off, LANES)]
        lab = p & 1
        e2 = (p >> 1) & 0x7FF
        rel = (p >> 12) & 0xF
        e1 = p >> 16
        flat = (rel << 22) + (e1 << 11) + e2
        idx_v[pl.ds(off, LANES)] = flat >> 1
        par_v[pl.ds(off, LANES)] = flat & 1
        sf_v[pl.ds(off, LANES)] = (2 * lab - 1).astype(jnp.float32)
        return carry

    lax.fori_loop(0, STEPS, idx_body, 0)

    pltpu.async_copy(table_hbm.at[idx_v], vals_v, sem).wait()

    def acc_body(i, carry):
        accp, accn = carry
        off = i * LANES
        w = vals_v[pl.ds(off, LANES)]
        par = par_v[pl.ds(off, LANES)]
        xbits = (w << ((1 - par) * 16)) & jnp.int32(-65536)
        x = lax.bitcast_convert_type(xbits, jnp.float32)
        sf = sf_v[pl.ds(off, LANES)]
        p = 1.0 / (1.0 + jnp.exp(-(sf * x)))
        return accp + p, accn + (1.0 - sf) * 0.5

    zeros = jnp.zeros((LANES,), jnp.float32)
    accp, accn = lax.fori_loop(0, STEPS, acc_body, (zeros, zeros))

    accp_v[...] = accp
    accn_v[...] = accn
    pltpu.sync_copy(accp_v, outp_hbm.at[wid])
    pltpu.sync_copy(accn_v, outn_hbm.at[wid])


@functools.partial(
    pl.kernel,
    out_type=(
        jax.ShapeDtypeStruct((NUM_WORKERS, LANES), jnp.float32),
        jax.ShapeDtypeStruct((NUM_WORKERS, LANES), jnp.float32),
    ),
    mesh=plsc.VectorSubcoreMesh(
        core_axis_name="c", subcore_axis_name="s",
        num_cores=NUM_CORES, num_subcores=NUM_SUBCORES,
    ),
    scratch_types=[
        pltpu.VMEM((B,), jnp.int32),
        pltpu.VMEM((B,), jnp.int32),
        pltpu.VMEM((B,), jnp.int32),
        pltpu.VMEM((B,), jnp.float32),
        pltpu.VMEM((B,), jnp.int32),
        pltpu.VMEM((LANES,), jnp.float32),
        pltpu.VMEM((LANES,), jnp.float32),
        pltpu.SemaphoreType.DMA,
    ],
)
def _sc_loss(*refs):
    _sc_body(*refs)


def kernel(predicted_values, labels):
    lab2 = labels.astype(jnp.int32)
    packed = ((lab2[:, 0] << 16) | (lab2[:, 1] << 12)
              | (lab2[:, 2] << 1) | lab2[:, 3])
    table_bf = predicted_values.astype(jnp.bfloat16).reshape(-1)
    table_words = lax.bitcast_convert_type(
        table_bf.reshape(R * N * N // 2, 2), jnp.int32)
    partial_p, partial_n = _sc_loss(table_words, packed)
    sum_p = jnp.sum(partial_p)
    neg = jnp.sum(partial_n)
    loss = (-1.0 / ((1.0 + neg) * jnp.float32(L))) * sum_p
    return jnp.reshape(loss, (1,)).astype(jnp.float32)


# speedup vs baseline: 101.9105x; 101.9105x over previous
import functools

import jax
import jax.numpy as jnp
from jax import lax
from jax.experimental import pallas as pl
from jax.experimental.pallas import tpu as pltpu
from jax.experimental.pallas import tpu_sc as plsc

R = 16
N = 2048
L = 262144

NUM_CORES = 2
NUM_SUBCORES = 16
NUM_WORKERS = NUM_CORES * NUM_SUBCORES
B = L // NUM_WORKERS
LANES = 16
STEPS = B // LANES


def _sc_body(table_hbm, packed_hbm, outp_hbm, outn_hbm,
             packed_v, idx_v, sf_v, vals_v, accp_v, accn_v, sem):
    wid = lax.axis_index("s") * NUM_CORES + lax.axis_index("c")
    base = wid * B

    pltpu.sync_copy(packed_hbm.at[pl.ds(base, B)], packed_v)

    def idx_body(i, carry):
        off = i * LANES
        p = packed_v[pl.ds(off, LANES)]
        lab = p & 1
        e2 = (p >> 1) & 0x7FF
        rel = (p >> 12) & 0xF
        e1 = p >> 16
        idx_v[pl.ds(off, LANES)] = (rel << 22) + (e1 << 11) + e2
        sf_v[pl.ds(off, LANES)] = (2 * lab - 1).astype(jnp.float32)
        return carry

    lax.fori_loop(0, STEPS, idx_body, 0)

    pltpu.async_copy(table_hbm.at[idx_v], vals_v, sem).wait()

    def acc_body(i, carry):
        accp, accn = carry
        off = i * LANES
        x = vals_v[pl.ds(off, LANES)]
        sf = sf_v[pl.ds(off, LANES)]
        p = 1.0 / (1.0 + jnp.exp(-(sf * x)))
        return accp + p, accn + (1.0 - sf) * 0.5

    zeros = jnp.zeros((LANES,), jnp.float32)
    accp, accn = lax.fori_loop(0, STEPS, acc_body, (zeros, zeros))

    accp_v[...] = accp
    accn_v[...] = accn
    pltpu.sync_copy(accp_v, outp_hbm.at[wid])
    pltpu.sync_copy(accn_v, outn_hbm.at[wid])


@functools.partial(
    pl.kernel,
    out_type=(
        jax.ShapeDtypeStruct((NUM_WORKERS, LANES), jnp.float32),
        jax.ShapeDtypeStruct((NUM_WORKERS, LANES), jnp.float32),
    ),
    mesh=plsc.VectorSubcoreMesh(
        core_axis_name="c", subcore_axis_name="s",
        num_cores=NUM_CORES, num_subcores=NUM_SUBCORES,
    ),
    scratch_types=[
        pltpu.VMEM((B,), jnp.int32),
        pltpu.VMEM((B,), jnp.int32),
        pltpu.VMEM((B,), jnp.float32),
        pltpu.VMEM((B,), jnp.float32),
        pltpu.VMEM((LANES,), jnp.float32),
        pltpu.VMEM((LANES,), jnp.float32),
        pltpu.SemaphoreType.DMA,
    ],
)
def _sc_loss(*refs):
    _sc_body(*refs)


def kernel(predicted_values, labels):
    lab2 = labels.astype(jnp.int32)
    packed = ((lab2[:, 0] << 16) | (lab2[:, 1] << 12)
              | (lab2[:, 2] << 1) | lab2[:, 3])
    partial_p, partial_n = _sc_loss(predicted_values.reshape(-1), packed)
    sum_p = jnp.sum(partial_p)
    neg = jnp.sum(partial_n)
    loss = (-1.0 / ((1.0 + neg) * jnp.float32(L))) * sum_p
    return jnp.reshape(loss, (1,)).astype(jnp.float32)
